# baseline (speedup 1.0000x reference)
import functools

import jax
import jax.numpy as jnp
from jax.experimental import pallas as pl
from jax.experimental.pallas import tpu as pltpu


def _fused_body(x_ref, w1_ref, b1_ref, w2_ref, b2_ref, o_ref, acc_ref,
                *, n_seq, s_tile, even):
    s = pl.program_id(1)

    @pl.when(s == 0)
    def _zero():
        acc_ref[...] = jnp.zeros_like(acc_ref)

    x = x_ref[...].astype(jnp.float32)
    if not even:
        pos = s * s_tile + jax.lax.broadcasted_iota(jnp.int32, x.shape, 1)
        x = jnp.where(pos < n_seq, x, 0.0)
    acc_ref[...] += jnp.sum(x, axis=1)

    @pl.when(s == pl.num_programs(1) - 1)
    def _head():
        pooled = acc_ref[...] * (1.0 / n_seq)
        h = jnp.dot(pooled, w1_ref[...].astype(jnp.float32),
                    preferred_element_type=jnp.float32)
        h = jnp.tanh(h + b1_ref[...].astype(jnp.float32))
        logits = jnp.dot(h, w2_ref[...].astype(jnp.float32),
                         preferred_element_type=jnp.float32)
        o_ref[...] = (logits + b2_ref[...].astype(jnp.float32)).astype(o_ref.dtype)


def _pick_seq_tile(S, max_rows):
    if S <= max_rows:
        return S
    best = 0
    for t in range(8, max_rows + 1, 8):
        if S % t == 0:
            best = t
    if best >= 64 or best >= max_rows // 2:
        return best
    return (max_rows // 8) * 8


def kernel(hidden_states, w_dense, b_dense, w_out, b_out):
    B, S, H = hidden_states.shape
    L = w_out.shape[1]
    x_bytes = jnp.dtype(hidden_states.dtype).itemsize

    L_pad = (L + 127) // 128 * 128
    if L_pad != L:
        w_out = jnp.pad(w_out, ((0, 0), (0, L_pad - L)))
        b_out = jnp.pad(b_out, ((0, 0), (0, L_pad - L)))

    B_t = B if B <= 64 else 64
    grid_b = pl.cdiv(B, B_t)
    B_out = grid_b * B_t

    w_bytes = (H * H + H + H * L_pad + L_pad) * 4
    fixed = w_bytes + B_t * H * 4 + 2 * B_t * L_pad * 4
    budget = int((64 << 20) * 0.82) - fixed - (2 << 20)
    max_rows = max(8, min((budget // 2) // (B_t * H * x_bytes), 64))
    S_t = _pick_seq_tile(S, max_rows)
    grid_s = pl.cdiv(S, S_t)
    even = (S % S_t == 0)

    body = functools.partial(_fused_body, n_seq=S, s_tile=S_t, even=even)

    def _const(shape):
        return pl.BlockSpec(shape, lambda b, s: (0, 0),
                            pipeline_mode=pl.Buffered(1))

    vmem_limit = min(2 * B_t * S_t * H * x_bytes + fixed + (4 << 20),
                     int((64 << 20) * 0.92))

    out = pl.pallas_call(
        body,
        out_shape=jax.ShapeDtypeStruct((B_out, L_pad), jnp.float32),
        grid_spec=pltpu.PrefetchScalarGridSpec(
            num_scalar_prefetch=0,
            grid=(grid_b, grid_s),
            in_specs=[
                pl.BlockSpec((B_t, S_t, H), lambda b, s: (b, s, 0)),
                _const((H, H)),
                _const((1, H)),
                _const((H, L_pad)),
                _const((1, L_pad)),
            ],
            out_specs=pl.BlockSpec((B_t, L_pad), lambda b, s: (b, 0)),
            scratch_shapes=[pltpu.VMEM((B_t, H), jnp.float32)],
        ),
        compiler_params=pltpu.CompilerParams(
            dimension_semantics=("parallel", "arbitrary"),
            vmem_limit_bytes=int(vmem_limit),
        ),
        cost_estimate=pl.CostEstimate(
            flops=int(B * S * H + 2 * B_out * H * (H + L_pad)),
            transcendentals=int(B_out * H),
            bytes_accessed=int(B * S * H * x_bytes + w_bytes
                               + B_out * L_pad * 4),
        ),
    )(hidden_states, w_dense, b_dense, w_out, b_out)

    return out[:B, :L]


# speedup vs baseline: 1.0291x; 1.0291x over previous
import functools

import jax
import jax.numpy as jnp
from jax.experimental import pallas as pl
from jax.experimental.pallas import tpu as pltpu


def _fused_body(xa_ref, xb_ref, w1_ref, b1_ref, w2_ref, b2_ref, o_ref,
                acc_ref, *, n_seq):
    s = pl.program_id(1)

    @pl.when(s == 0)
    def _zero():
        acc_ref[...] = jnp.zeros_like(acc_ref)

    acc_ref[...] += (jnp.sum(xa_ref[...].astype(jnp.float32), axis=1)
                     + jnp.sum(xb_ref[...].astype(jnp.float32), axis=1))

    @pl.when(s == pl.num_programs(1) - 1)
    def _head():
        pooled = acc_ref[...] * (1.0 / n_seq)
        h = jnp.dot(pooled, w1_ref[...].astype(jnp.float32),
                    preferred_element_type=jnp.float32)
        h = jnp.tanh(h + b1_ref[...].astype(jnp.float32))
        logits = jnp.dot(h, w2_ref[...].astype(jnp.float32),
                         preferred_element_type=jnp.float32)
        o_ref[...] = (logits + b2_ref[...].astype(jnp.float32)).astype(o_ref.dtype)


def _pick_split_tile(S, max_rows):
    best = 0
    for t in range(8, max_rows + 1, 8):
        if S % (2 * t) == 0:
            best = t
    return best


def kernel(hidden_states, w_dense, b_dense, w_out, b_out):
    B, S, H = hidden_states.shape
    L = w_out.shape[1]
    x_bytes = jnp.dtype(hidden_states.dtype).itemsize

    L_pad = (L + 127) // 128 * 128
    if L_pad != L:
        w_out = jnp.pad(w_out, ((0, 0), (0, L_pad - L)))
        b_out = jnp.pad(b_out, ((0, 0), (0, L_pad - L)))

    B_t = B if B <= 64 else 64
    grid_b = pl.cdiv(B, B_t)
    B_out = grid_b * B_t

    w_bytes = (H * H + H + H * L_pad + L_pad) * 4
    fixed = w_bytes + B_t * H * 4 + 2 * B_t * L_pad * 4
    budget = int((64 << 20) * 0.82) - fixed - (2 << 20)
    max_rows = max(8, (budget // 4) // (B_t * H * x_bytes))
    S_t = _pick_split_tile(S, max_rows)
    S_true = S
    if S_t == 0:
        S_t = max(8, min(max_rows, 64))
        S_padded = (S + 2 * S_t - 1) // (2 * S_t) * (2 * S_t)
        hidden_states = jnp.pad(hidden_states,
                                ((0, 0), (0, S_padded - S), (0, 0)))
        S = S_padded
    grid_s = S // (2 * S_t)

    body = functools.partial(_fused_body, n_seq=S_true)

    def _const(shape):
        return pl.BlockSpec(shape, lambda b, s: (0, 0),
                            pipeline_mode=pl.Buffered(1))

    vmem_limit = min(4 * B_t * S_t * H * x_bytes + fixed + (4 << 20),
                     int((64 << 20) * 0.92))

    out = pl.pallas_call(
        body,
        out_shape=jax.ShapeDtypeStruct((B_out, L_pad), jnp.float32),
        grid_spec=pltpu.PrefetchScalarGridSpec(
            num_scalar_prefetch=0,
            grid=(grid_b, grid_s),
            in_specs=[
                pl.BlockSpec((B_t, S_t, H), lambda b, s: (b, 2 * s, 0)),
                pl.BlockSpec((B_t, S_t, H), lambda b, s: (b, 2 * s + 1, 0)),
                _const((H, H)),
                _const((1, H)),
                _const((H, L_pad)),
                _const((1, L_pad)),
            ],
            out_specs=pl.BlockSpec((B_t, L_pad), lambda b, s: (b, 0)),
            scratch_shapes=[pltpu.VMEM((B_t, H), jnp.float32)],
        ),
        compiler_params=pltpu.CompilerParams(
            dimension_semantics=("parallel", "arbitrary"),
            vmem_limit_bytes=int(vmem_limit),
        ),
        cost_estimate=pl.CostEstimate(
            flops=int(B * S * H + 2 * B_out * H * (H + L_pad)),
            transcendentals=int(B_out * H),
            bytes_accessed=int(B * S * H * x_bytes + w_bytes
                               + B_out * L_pad * 4),
        ),
    )(hidden_states, hidden_states, w_dense, b_dense, w_out, b_out)

    return out[:B, :L]
